# baseline (speedup 1.0000x reference)
import functools

import jax
import jax.numpy as jnp
from jax import lax
from jax.experimental import pallas as pl
from jax.experimental.pallas import tpu as pltpu
from jax.experimental.pallas import tpu_sc as plsc

B = 16384
ND = 13
NS = 26
E = 16
V = 100000
D0 = (ND + NS) * E
H1 = 512
H2 = 256
DE = ND * E
SE = NS * E

NC = 2
NSUB = 16
NW = NC * NSUB
ROWS = B * NS
RPW = ROWS // NW
CH = 832
NCHUNK = RPW // CH
SUBG = 128
NSUBG = CH // SUBG + 1


GR = NS * V // 8
BAGS = CH // NS
NG16 = CH // 16


def _fix_indices(idx_v, sub_v, fv_v):
    @plsc.parallel_loop(0, CH // 16, unroll=4)
    def _fix(i):
        g = idx_v[pl.ds(i * 16, 16)] + fv_v[pl.ds(i * 16, 16)]
        idx_v[pl.ds(i * 16, 16)] = lax.shift_right_logical(g, 3)
        sub_v[pl.ds(i * 16, 16)] = lax.bitwise_and(g, 7)


def _gather_chunk(tab, idx_v, rows_v, sem):
    offs = list(range(0, CH, SUBG))
    cps = [
        pltpu.async_copy(
            tab.at[idx_v.at[pl.ds(o, min(SUBG, CH - o))]],
            rows_v.at[pl.ds(o, min(SUBG, CH - o))],
            sem,
        )
        for o in offs
    ]
    for cp in cps:
        cp.wait()


def _select_rows(rows_v, sub_v, sel_v, bagl_v, lanel_v, lane):
    @plsc.parallel_loop(0, NG16, unroll=1)
    def _sel(i):
        r0 = i * 16
        rvec = r0 + lane
        subs = sub_v[pl.ds(r0, 16)]
        bagv = bagl_v[pl.ds(r0, 16)]
        lanev = lanel_v[pl.ds(r0, 16)]
        soff = lax.shift_left(subs, 4)
        for k in range(E):
            col = plsc.load_gather(rows_v, [rvec, soff + k])
            plsc.store_scatter(sel_v, [bagv, lanev + k], col)


def _sc_gather_tab_body(tab, idxh, fvh, baglh, lanelh, out,
                        idx_v, sub_v, fv_v, bagl_v, lanel_v, rows_v, sel_v,
                        sem):
    c = lax.axis_index("c")
    s = lax.axis_index("s")
    base_w = (s * NC + c) * RPW
    lane = lax.iota(jnp.int32, 16)
    pltpu.sync_copy(baglh, bagl_v)
    pltpu.sync_copy(lanelh, lanel_v)

    def chunk(k, carry):
        base = pl.multiple_of(base_w + k * CH, 8)
        pltpu.sync_copy(idxh.at[pl.ds(base, CH)], idx_v)
        pltpu.sync_copy(fvh.at[pl.ds(base, CH)], fv_v)
        _fix_indices(idx_v, sub_v, fv_v)
        _gather_chunk(tab, idx_v, rows_v, sem)
        _select_rows(rows_v, sub_v, sel_v, bagl_v, lanel_v, lane)
        pltpu.sync_copy(sel_v, out.at[pl.ds(pl.multiple_of(base // NS, 8), BAGS)])
        return carry

    lax.fori_loop(0, NCHUNK, chunk, 0)


def _sc_mesh():
    return plsc.VectorSubcoreMesh(
        core_axis_name="c", subcore_axis_name="s",
        num_cores=NC, num_subcores=NSUB,
    )


@functools.cache
def _get_sc_gather_tab():
    return functools.partial(
        pl.kernel,
        out_type=jax.ShapeDtypeStruct((B, 512), jnp.float32),
        mesh=_sc_mesh(),
        scratch_types=[
            pltpu.VMEM((CH,), jnp.int32),
            pltpu.VMEM((CH,), jnp.int32),
            pltpu.VMEM((CH,), jnp.int32),
            pltpu.VMEM((CH,), jnp.int32),
            pltpu.VMEM((CH,), jnp.int32),
            pltpu.VMEM((CH, 128), jnp.float32),
            pltpu.VMEM((BAGS, 512), jnp.float32),
            pltpu.SemaphoreType.DMA,
        ],
        compiler_params=pltpu.CompilerParams(
            use_tc_tiling_on_sc=True, needs_layout_passes=False),
    )(_sc_gather_tab_body)


BK = 1024
G = B // BK


def _expand_mat(wf):
    d = lax.broadcasted_iota(jnp.int32, (ND, DE), 0)
    j = lax.broadcasted_iota(jnp.int32, (ND, DE), 1)
    return jnp.where(j // E == d, jnp.broadcast_to(wf, (ND, DE)), 0.0)


def _k1_body(xi, w1f, b1f, co1):
    w1e = _expand_mat(w1f[0:1, :])
    co1[...] = (jnp.dot(xi[...], w1e, preferred_element_type=jnp.float32, precision=lax.Precision.HIGHEST)
                + b1f[0:1, :])


def _make_k1(interpret=False):
    return pl.pallas_call(
        _k1_body,
        grid=(G,),
        in_specs=[
            pl.BlockSpec((BK, ND), lambda i: (i, 0)),
            pl.BlockSpec((8, DE), lambda i: (0, 0)),
            pl.BlockSpec((8, DE), lambda i: (0, 0)),
        ],
        out_specs=pl.BlockSpec((BK, DE), lambda i: (i, 0)),
        out_shape=jax.ShapeDtypeStruct((B, DE), jnp.float32),
        interpret=interpret,
    )


def _p1_body(e1, e2, w1, y, xvd, xi, w2f, b2f, l1w, l1b,
             sfm, h1, hsum, hsq):
    i = pl.program_id(0)
    f32 = jnp.float32
    E1 = e1[...][:, 0:SE]
    W1 = w1[...]
    md = (lax.broadcasted_iota(jnp.int32, (DE, ND), 0) // E
          == lax.broadcasted_iota(jnp.int32, (DE, ND), 1)).astype(f32)
    ys = jnp.dot(y[...], md, preferred_element_type=f32, precision=lax.Precision.HIGHEST)
    s1 = jnp.sum(E1 * W1, axis=1) + jnp.sum(ys * xvd[...], axis=1)
    w2e = _expand_mat(w2f[0:1, :])
    co2 = jnp.dot(xi[...], w2e, preferred_element_type=f32, precision=lax.Precision.HIGHEST) + b2f[0:1, :]
    P2 = e2[...][:, 0:SE] * W1
    r208 = lax.broadcasted_iota(jnp.int32, (DE, E), 0) % E
    m208 = (r208 == lax.broadcasted_iota(jnp.int32, (DE, E), 1)).astype(jnp.float32)
    r416 = lax.broadcasted_iota(jnp.int32, (SE, E), 0) % E
    m416 = (r416 == lax.broadcasted_iota(jnp.int32, (SE, E), 1)).astype(jnp.float32)
    s = (jnp.dot(co2, m208, preferred_element_type=f32, precision=lax.Precision.HIGHEST)
         + jnp.dot(P2, m416, preferred_element_type=f32, precision=lax.Precision.HIGHEST))
    sqs = (jnp.dot(co2 * co2, m208, preferred_element_type=f32, precision=lax.Precision.HIGHEST)
           + jnp.dot(P2 * P2, m416, preferred_element_type=f32, precision=lax.Precision.HIGHEST))
    S2 = 0.5 * jnp.sum(s * s - sqs, axis=1)
    sfm[...] = s1 + S2
    h = (jnp.dot(co2, l1w[0:DE, :], preferred_element_type=f32)
         + jnp.dot(P2, l1w[DE:D0, :], preferred_element_type=f32)
         + l1b[...])
    h1[...] = h

    @pl.when(i == 0)
    def _():
        hsum[...] = jnp.zeros_like(hsum)
        hsq[...] = jnp.zeros_like(hsq)

    hsum[...] += jnp.sum(h, axis=0)
    hsq[...] += jnp.sum(h * h, axis=0)


def _make_p1(interpret=False):
    full512 = pl.BlockSpec((H1,), lambda i: (0,))
    return pl.pallas_call(
        _p1_body,
        grid=(G,),
        in_specs=[
            pl.BlockSpec((BK, 512), lambda i: (i, 0)),
            pl.BlockSpec((BK, 512), lambda i: (i, 0)),
            pl.BlockSpec((BK, SE), lambda i: (i, 0)),
            pl.BlockSpec((BK, DE), lambda i: (i, 0)),
            pl.BlockSpec((BK, ND), lambda i: (i, 0)),
            pl.BlockSpec((BK, ND), lambda i: (i, 0)),
            pl.BlockSpec((8, DE), lambda i: (0, 0)),
            pl.BlockSpec((8, DE), lambda i: (0, 0)),
            pl.BlockSpec((D0, H1), lambda i: (0, 0)),
            full512,
        ],
        out_specs=[
            pl.BlockSpec((BK,), lambda i: (i,)),
            pl.BlockSpec((BK, H1), lambda i: (i, 0)),
            full512,
            full512,
        ],
        out_shape=[
            jax.ShapeDtypeStruct((B,), jnp.float32),
            jax.ShapeDtypeStruct((B, H1), jnp.float32),
            jax.ShapeDtypeStruct((H1,), jnp.float32),
            jax.ShapeDtypeStruct((H1,), jnp.float32),
        ],
        interpret=interpret,
    )


def _p2_body(h1, hsum, hsq, g1, l2w, u, usum, usq):
    i = pl.program_id(0)
    mu = hsum[...] * (1.0 / B)
    va = hsq[...] * (1.0 / B) - mu * mu
    a1 = g1[...] * lax.rsqrt(va + 1e-5)
    zn = (h1[...] - mu) * a1
    uu = jnp.dot(zn, l2w[...], preferred_element_type=jnp.float32)
    u[...] = uu

    @pl.when(i == 0)
    def _():
        usum[...] = jnp.zeros_like(usum)
        usq[...] = jnp.zeros_like(usq)

    usum[...] += jnp.sum(uu, axis=0)
    usq[...] += jnp.sum(uu * uu, axis=0)


def _make_p2(interpret=False):
    full512 = pl.BlockSpec((H1,), lambda i: (0,))
    full256 = pl.BlockSpec((H2,), lambda i: (0,))
    return pl.pallas_call(
        _p2_body,
        grid=(G,),
        in_specs=[
            pl.BlockSpec((BK, H1), lambda i: (i, 0)),
            full512,
            full512,
            full512,
            pl.BlockSpec((H1, H2), lambda i: (0, 0)),
        ],
        out_specs=[
            pl.BlockSpec((BK, H2), lambda i: (i, 0)),
            full256,
            full256,
        ],
        out_shape=[
            jax.ShapeDtypeStruct((B, H2), jnp.float32),
            jax.ShapeDtypeStruct((H2,), jnp.float32),
            jax.ShapeDtypeStruct((H2,), jnp.float32),
        ],
        interpret=interpret,
    )


def _p3_body(u, usum, usq, g2, bb2, sfm, bias, out):
    mu = usum[...] * (1.0 / B)
    va = usq[...] * (1.0 / B) - mu * mu
    a2 = g2[...] * lax.rsqrt(va + 1e-5)
    c3 = jnp.sum(bb2[...]) - jnp.sum(mu * a2)
    s3 = jnp.sum(u[...] * a2, axis=1) + c3
    out[...] = sfm[...] + s3 + bias[...]


def _make_p3(interpret=False):
    full256 = pl.BlockSpec((H2,), lambda i: (0,))
    vec = pl.BlockSpec((BK,), lambda i: (i,))
    return pl.pallas_call(
        _p3_body,
        grid=(G,),
        in_specs=[
            pl.BlockSpec((BK, H2), lambda i: (i, 0)),
            full256,
            full256,
            full256,
            full256,
            vec,
            vec,
        ],
        out_specs=vec,
        out_shape=jax.ShapeDtypeStruct((B,), jnp.float32),
        interpret=interpret,
    )


_k1 = _make_k1()
_p1 = _make_p1()
_p2 = _make_p2()
_p3 = _make_p3()


@jax.jit
def kernel(Xi, Xv, conv1_W, conv1_b, conv2_W, conv2_b, emb1, emb2,
           lin1_W, lin1_b, bn1_g, bn1_b, lin2_W, lin2_b, bn2_g, bn2_b, bias):
    Xi_lin = Xi[:, :ND, 0].astype(jnp.float32)
    idx_flat = Xi[:, ND:, 0].reshape(-1)
    W1 = jnp.broadcast_to(Xv[:, ND:].reshape(NS, 1, B), (NS, E, B)).reshape(B, SE)
    XvD = Xv[:, :ND]
    w1f = jnp.broadcast_to(conv1_W.reshape(1, DE), (8, DE))
    b1f = jnp.broadcast_to(conv1_b.reshape(1, DE), (8, DE))
    w2f = jnp.broadcast_to(conv2_W.reshape(1, DE), (8, DE))
    b2f = jnp.broadcast_to(conv2_b.reshape(1, DE), (8, DE))

    ar = jnp.arange(ROWS, dtype=jnp.int32)
    fv = (ar % NS) * V
    arc = jnp.arange(CH, dtype=jnp.int32)
    bagl = arc // NS
    lanel = (arc % NS) * E
    E1s = _get_sc_gather_tab()(emb1.reshape(GR, 128), idx_flat, fv, bagl, lanel)
    E2s = _get_sc_gather_tab()(emb2.reshape(GR, 128), idx_flat, fv, bagl, lanel)

    co1 = _k1(Xi_lin, w1f, b1f)
    Y = co1.T.reshape(B, DE)

    sfm, h1, hsum, hsq = _p1(E1s, E2s, W1, Y, XvD, Xi_lin, w2f, b2f,
                             lin1_W, lin1_b)
    u, usum, usq = _p2(h1, hsum, hsq, bn1_g, lin2_W)
    return _p3(u, usum, usq, bn2_g, bn2_b, sfm, bias)
